# baseline (speedup 1.0000x reference)
import functools

import jax
import jax.numpy as jnp
import numpy as np
from jax.experimental import pallas as pl
from jax.experimental.pallas import tpu as pltpu

CIN = 256
PL = 128
COUT = 512
H = W = 28
HW = H * W
S = 2
HO = WO = 14
HPP = 15
PHROWS = 16 * HPP
NROWS = 4 * PHROWS
MROWS = HPP * HPP
MOUT = HO * WO


def _consts():
    p = np.zeros((HW, NROWS), np.float32)
    for ph in range(4):
        a, b = ph // 2, ph % 2
        for i in range(HPP):
            for j in range(HPP):
                h, w = 2 * i + a - 1, 2 * j + b - 1
                if 0 <= h < H and 0 <= w < W:
                    p[h * W + w, ph * PHROWS + HPP * i + j] = 1.0
    q = np.zeros((HW, MROWS), np.float32)
    for i in range(HPP):
        for j in range(HPP):
            h, w = 2 * i, 2 * j
            if h < H and w < W:
                q[h * W + w, HPP * i + j] = 1.0
    sc = np.zeros((MROWS, MOUT), np.float32)
    for i in range(HO):
        for j in range(WO):
            sc[HPP * i + j, i * WO + j] = 1.0
    return (jnp.asarray(p, jnp.bfloat16), jnp.asarray(q, jnp.bfloat16),
            jnp.asarray(sc))


def _body(x_ref, w1_ref, b1_ref, w2_ref, b2_ref, w3_ref, ws_ref, bsum_ref,
          p_ref, q_ref, sc_ref, o_ref, o1_scr):
    ta = (((0,), (0,)), ((), ()))
    f32 = jnp.float32
    xm = x_ref[0].astype(jnp.bfloat16)

    acc1 = jax.lax.dot_general(xm, w1_ref[...], ta, preferred_element_type=f32)
    o1 = jnp.maximum(acc1 + b1_ref[...], 0.0).astype(jnp.bfloat16)
    o1ph = jax.lax.dot_general(p_ref[...], o1, ta, preferred_element_type=f32)
    o1_scr[...] = o1ph.astype(jnp.bfloat16)

    cols = []
    for kh in range(3):
        for kw in range(3):
            ph = (kh % 2) * 2 + (kw % 2)
            base = ph * PHROWS + (kh // 2) * HPP + (kw // 2)
            cols.append(o1_scr[pl.ds(base, MROWS), :])
    patches = jnp.concatenate(cols, axis=1)
    acc2 = jnp.dot(patches, w2_ref[...], preferred_element_type=f32)
    o2 = jnp.maximum(acc2 + b2_ref[...], 0.0).astype(jnp.bfloat16)

    xs_cm = jnp.dot(xm, q_ref[...], preferred_element_type=f32)
    xs_cm = xs_cm.astype(jnp.bfloat16)

    acc3 = jnp.dot(o2, w3_ref[...], preferred_element_type=f32)
    accs = jax.lax.dot_general(xs_cm, ws_ref[...], ta,
                               preferred_element_type=f32)
    o3 = jnp.maximum(acc3 + accs + bsum_ref[...], 0.0)

    o_ref[0] = jax.lax.dot_general(o3, sc_ref[...], ta,
                                   preferred_element_type=f32)


def kernel(x, w1_p, b1_p, w2_p, b2_p, w3_p, b3_p, ws_p, bs_p):
    n = x.shape[0]
    pmat, qmat, sc = _consts()
    xrows = x.reshape(n, CIN, HW)

    w1 = w1_p.astype(jnp.bfloat16)
    w2 = w2_p.astype(jnp.bfloat16)
    w3 = w3_p.astype(jnp.bfloat16)
    ws = ws_p.astype(jnp.bfloat16)
    b1 = b1_p.reshape(1, PL)
    b2 = b2_p.reshape(1, PL)
    bsum = (b3_p + bs_p).reshape(1, COUT)

    flops = 2 * n * (HW * CIN * PL + HW * NROWS * PL + MROWS * 9 * PL * PL
                     + CIN * HW * MROWS + MROWS * PL * COUT
                     + MROWS * CIN * COUT + MROWS * COUT * MOUT)
    bytes_accessed = (n * HW * CIN * 4 + n * COUT * MOUT * 4
                      + (w1.size + w2.size + w3.size + ws.size) * 2)
    const = lambda i: (0, 0)
    out = pl.pallas_call(
        _body,
        grid=(n,),
        in_specs=[
            pl.BlockSpec((1, CIN, HW), lambda i: (i, 0, 0)),
            pl.BlockSpec((CIN, PL), const),
            pl.BlockSpec((1, PL), const),
            pl.BlockSpec((9 * PL, PL), const),
            pl.BlockSpec((1, PL), const),
            pl.BlockSpec((PL, COUT), const),
            pl.BlockSpec((CIN, COUT), const),
            pl.BlockSpec((1, COUT), const),
            pl.BlockSpec((HW, NROWS), const),
            pl.BlockSpec((HW, MROWS), const),
            pl.BlockSpec((MROWS, MOUT), const),
        ],
        out_specs=pl.BlockSpec((1, COUT, MOUT), lambda i: (i, 0, 0)),
        out_shape=jax.ShapeDtypeStruct((n, COUT, MOUT), jnp.float32),
        scratch_shapes=[pltpu.VMEM((NROWS, PL), jnp.bfloat16)],
        compiler_params=pltpu.CompilerParams(
            dimension_semantics=("parallel",),
            vmem_limit_bytes=64 * 1024 * 1024),
        cost_estimate=pl.CostEstimate(flops=flops, transcendentals=0,
                                      bytes_accessed=bytes_accessed),
    )(xrows, w1, b1, w2, b2, w3, ws, bsum, pmat, qmat, sc)
    return out.reshape(n, COUT, HO, WO)


# speedup vs baseline: 1.1068x; 1.1068x over previous
import functools

import jax
import jax.numpy as jnp
import numpy as np
from jax.experimental import pallas as pl
from jax.experimental.pallas import tpu as pltpu

CIN = 256
PL = 128
COUT = 512
H = W = 28
HW = H * W
HO = WO = 14
NPIX = HO * WO
HPP = 15
PHROWS = 16 * HPP
NROWS = 4 * PHROWS
MROWS = HPP * HPP
MOUT = HO * WO


def _consts():
    pp = np.zeros((4, NPIX, PHROWS), np.float32)
    for ph in range(4):
        a, b = ph // 2, ph % 2
        for i in range(HPP):
            for j in range(HPP):
                isrc, jsrc = i - (1 - a), j - (1 - b)
                if 0 <= isrc < HO and 0 <= jsrc < WO:
                    pp[ph, isrc * WO + jsrc, HPP * i + j] = 1.0
    sc = np.zeros((MROWS, MOUT), np.float32)
    for i in range(HO):
        for j in range(WO):
            sc[HPP * i + j, i * WO + j] = 1.0
    return jnp.asarray(pp, jnp.bfloat16), jnp.asarray(sc, jnp.bfloat16)


def _body(x_ref, w1_ref, b1_ref, w2_ref, b2_ref, w3_ref, ws_ref, bsum_ref,
          pp_ref, sc_ref, o_ref, o1_scr):
    ta = (((0,), (0,)), ((), ()))
    f32 = jnp.float32
    xm = x_ref[0]

    acc1 = jnp.dot(xm, w1_ref[...], preferred_element_type=f32)
    o1 = jnp.maximum(acc1 + b1_ref[...], 0.0).astype(jnp.bfloat16)

    for ph in range(4):
        a, b = ph // 2, ph % 2
        src = ((a + 1) % 2) * 2 + ((b + 1) % 2)
        o1p = jax.lax.dot_general(pp_ref[ph], o1[src * NPIX:(src + 1) * NPIX],
                                  ta, preferred_element_type=f32)
        o1_scr[pl.ds(ph * PHROWS, PHROWS), :] = o1p.astype(jnp.bfloat16)

    cols = []
    for kh in range(3):
        for kw in range(3):
            ph = (kh % 2) * 2 + (kw % 2)
            base = ph * PHROWS + (kh // 2) * HPP + (kw // 2)
            cols.append(o1_scr[pl.ds(base, MROWS), :])
    patches = jnp.concatenate(cols, axis=1)
    acc2 = jnp.dot(patches, w2_ref[...], preferred_element_type=f32)
    o2 = jnp.maximum(acc2 + b2_ref[...], 0.0).astype(jnp.bfloat16)
    o2c = jax.lax.dot_general(sc_ref[...], o2, ta,
                              preferred_element_type=f32).astype(jnp.bfloat16)

    xs = xm[0:NPIX, :]
    acc3 = jnp.dot(o2c, w3_ref[...], preferred_element_type=f32)
    accs = jnp.dot(xs, ws_ref[...], preferred_element_type=f32)
    o3 = jnp.maximum(acc3 + accs + bsum_ref[...], 0.0)

    o_ref[0] = jnp.transpose(o3)


def kernel(x, w1_p, b1_p, w2_p, b2_p, w3_p, b3_p, ws_p, bs_p):
    n = x.shape[0]
    ppmat, sc = _consts()
    xr = x.reshape(n, CIN, HO, 2, WO, 2)
    xr = jnp.transpose(xr, (0, 3, 5, 2, 4, 1))
    xrows = xr.reshape(n, HW, CIN).astype(jnp.bfloat16)

    w1 = w1_p.astype(jnp.bfloat16)
    w2 = w2_p.astype(jnp.bfloat16)
    w3 = w3_p.astype(jnp.bfloat16)
    ws = ws_p.astype(jnp.bfloat16)
    b1 = b1_p.reshape(1, PL)
    b2 = b2_p.reshape(1, PL)
    bsum = (b3_p + bs_p).reshape(1, COUT)

    flops = 2 * n * (HW * CIN * PL + 4 * NPIX * PHROWS * PL
                     + MROWS * 9 * PL * PL + MROWS * MOUT * PL
                     + MOUT * PL * COUT + MOUT * CIN * COUT)
    bytes_accessed = (n * HW * CIN * 2 + n * COUT * MOUT * 4
                      + (w1.size + w2.size + w3.size + ws.size) * 2)
    const = lambda i: (0, 0)
    out = pl.pallas_call(
        _body,
        grid=(n,),
        in_specs=[
            pl.BlockSpec((1, HW, CIN), lambda i: (i, 0, 0)),
            pl.BlockSpec((CIN, PL), const),
            pl.BlockSpec((1, PL), const),
            pl.BlockSpec((9 * PL, PL), const),
            pl.BlockSpec((1, PL), const),
            pl.BlockSpec((PL, COUT), const),
            pl.BlockSpec((CIN, COUT), const),
            pl.BlockSpec((1, COUT), const),
            pl.BlockSpec((4, NPIX, PHROWS), lambda i: (0, 0, 0)),
            pl.BlockSpec((MROWS, MOUT), const),
        ],
        out_specs=pl.BlockSpec((1, COUT, MOUT), lambda i: (i, 0, 0)),
        out_shape=jax.ShapeDtypeStruct((n, COUT, MOUT), jnp.float32),
        scratch_shapes=[pltpu.VMEM((NROWS, PL), jnp.bfloat16)],
        compiler_params=pltpu.CompilerParams(
            dimension_semantics=("parallel",),
            vmem_limit_bytes=64 * 1024 * 1024),
        cost_estimate=pl.CostEstimate(flops=flops, transcendentals=0,
                                      bytes_accessed=bytes_accessed),
    )(xrows, w1, b1, w2, b2, w3, ws, bsum, ppmat, sc)
    return out.reshape(n, COUT, HO, WO)


# speedup vs baseline: 1.2063x; 1.0899x over previous
import functools

import jax
import jax.numpy as jnp
import numpy as np
from jax.experimental import pallas as pl
from jax.experimental.pallas import tpu as pltpu

CIN = 256
PL = 128
COUT = 512
H = W = 28
HW = H * W
HO = WO = 14
NPIX = HO * WO
HPP = 15
PHROWS = 16 * HPP
NROWS = 4 * PHROWS
MROWS = HPP * HPP
MOUT = HO * WO
BIMG = 2


def _consts():
    pp = np.zeros((4, NPIX, PHROWS), np.float32)
    for ph in range(4):
        a, b = ph // 2, ph % 2
        for i in range(HPP):
            for j in range(HPP):
                isrc, jsrc = i - (1 - a), j - (1 - b)
                if 0 <= isrc < HO and 0 <= jsrc < WO:
                    pp[ph, isrc * WO + jsrc, HPP * i + j] = 1.0
    sc = np.zeros((MROWS, MOUT), np.float32)
    for i in range(HO):
        for j in range(WO):
            sc[HPP * i + j, i * WO + j] = 1.0
    return jnp.asarray(pp, jnp.bfloat16), jnp.asarray(sc, jnp.bfloat16)


def _body(x_ref, w1_ref, b1_ref, w2_ref, b2_ref, w3_ref, ws_ref, bsum_ref,
          pp_ref, sc_ref, o_ref, o1_scr):
    ta = (((0,), (0,)), ((), ()))
    f32 = jnp.float32
    for bimg in range(BIMG):
        xm = x_ref[bimg]

        acc1 = jnp.dot(xm, w1_ref[...], preferred_element_type=f32)
        o1 = jnp.maximum(acc1 + b1_ref[...], 0.0).astype(jnp.bfloat16)

        for ph in range(4):
            a, b = ph // 2, ph % 2
            src = ((a + 1) % 2) * 2 + ((b + 1) % 2)
            o1p = jax.lax.dot_general(pp_ref[ph],
                                      o1[src * NPIX:(src + 1) * NPIX],
                                      ta, preferred_element_type=f32)
            o1_scr[bimg, pl.ds(ph * PHROWS, PHROWS), :] = (
                o1p.astype(jnp.bfloat16))

        cols = []
        for kh in range(3):
            for kw in range(3):
                ph = (kh % 2) * 2 + (kw % 2)
                base = ph * PHROWS + (kh // 2) * HPP + (kw // 2)
                cols.append(o1_scr[bimg, pl.ds(base, MROWS), :])
        patches = jnp.concatenate(cols, axis=1)
        acc2 = jnp.dot(patches, w2_ref[...], preferred_element_type=f32)
        o2 = jnp.maximum(acc2 + b2_ref[...], 0.0).astype(jnp.bfloat16)
        o2c = jax.lax.dot_general(
            sc_ref[...], o2, ta,
            preferred_element_type=f32).astype(jnp.bfloat16)

        xs = xm[0:NPIX, :]
        acc3 = jnp.dot(o2c, w3_ref[...], preferred_element_type=f32)
        accs = jnp.dot(xs, ws_ref[...], preferred_element_type=f32)
        o3 = jnp.maximum(acc3 + accs + bsum_ref[...], 0.0)

        o_ref[bimg] = jnp.transpose(o3).astype(jnp.bfloat16)


def kernel(x, w1_p, b1_p, w2_p, b2_p, w3_p, b3_p, ws_p, bs_p):
    n = x.shape[0]
    ppmat, sc = _consts()
    xr = x.astype(jnp.bfloat16).reshape(n, CIN, HO, 2, WO, 2)
    xr = jnp.transpose(xr, (0, 3, 5, 2, 4, 1))
    xrows = xr.reshape(n, HW, CIN)

    w1 = w1_p.astype(jnp.bfloat16)
    w2 = w2_p.astype(jnp.bfloat16)
    w3 = w3_p.astype(jnp.bfloat16)
    ws = ws_p.astype(jnp.bfloat16)
    b1 = b1_p.reshape(1, PL)
    b2 = b2_p.reshape(1, PL)
    bsum = (b3_p + bs_p).reshape(1, COUT)

    flops = 2 * n * (HW * CIN * PL + 4 * NPIX * PHROWS * PL
                     + MROWS * 9 * PL * PL + MROWS * MOUT * PL
                     + MOUT * PL * COUT + MOUT * CIN * COUT)
    bytes_accessed = (n * HW * CIN * 2 + n * COUT * MOUT * 4
                      + (w1.size + w2.size + w3.size + ws.size) * 2)
    const = lambda i: (0, 0)
    out = pl.pallas_call(
        _body,
        grid=(n // BIMG,),
        in_specs=[
            pl.BlockSpec((BIMG, HW, CIN), lambda i: (i, 0, 0)),
            pl.BlockSpec((CIN, PL), const),
            pl.BlockSpec((1, PL), const),
            pl.BlockSpec((9 * PL, PL), const),
            pl.BlockSpec((1, PL), const),
            pl.BlockSpec((PL, COUT), const),
            pl.BlockSpec((CIN, COUT), const),
            pl.BlockSpec((1, COUT), const),
            pl.BlockSpec((4, NPIX, PHROWS), lambda i: (0, 0, 0)),
            pl.BlockSpec((MROWS, MOUT), const),
        ],
        out_specs=pl.BlockSpec((BIMG, COUT, MOUT), lambda i: (i, 0, 0)),
        out_shape=jax.ShapeDtypeStruct((n, COUT, MOUT), jnp.bfloat16),
        scratch_shapes=[pltpu.VMEM((BIMG, NROWS, PL), jnp.bfloat16)],
        compiler_params=pltpu.CompilerParams(
            dimension_semantics=("parallel",),
            vmem_limit_bytes=64 * 1024 * 1024),
        cost_estimate=pl.CostEstimate(flops=flops, transcendentals=0,
                                      bytes_accessed=bytes_accessed),
    )(xrows, w1, b1, w2, b2, w3, ws, bsum, ppmat, sc)
    return out.astype(jnp.float32).reshape(n, COUT, HO, WO)


# speedup vs baseline: 1.2397x; 1.0277x over previous
import functools

import jax
import jax.numpy as jnp
import numpy as np
from jax.experimental import pallas as pl
from jax.experimental.pallas import tpu as pltpu

CIN = 256
PL = 128
COUT = 512
H = W = 28
HW = H * W
HO = WO = 14
NPIX = HO * WO
HPP = 15
PHROWS = 16 * HPP
NROWS = 4 * PHROWS
MROWS = HPP * HPP
MOUT = HO * WO
BIMG = 4


def _consts():
    pp = np.zeros((4, NPIX, PHROWS), np.float32)
    for ph in range(4):
        a, b = ph // 2, ph % 2
        for i in range(HPP):
            for j in range(HPP):
                isrc, jsrc = i - (1 - a), j - (1 - b)
                if 0 <= isrc < HO and 0 <= jsrc < WO:
                    pp[ph, isrc * WO + jsrc, HPP * i + j] = 1.0
    sc = np.zeros((MROWS, MOUT), np.float32)
    for i in range(HO):
        for j in range(WO):
            sc[HPP * i + j, i * WO + j] = 1.0
    return jnp.asarray(pp, jnp.bfloat16), jnp.asarray(sc, jnp.bfloat16)


def _body(x_ref, w1_ref, b1_ref, w2_ref, b2_ref, w3_ref, ws_ref, bsum_ref,
          pp_ref, sc_ref, o_ref, o1_scr):
    ta = (((0,), (0,)), ((), ()))
    f32 = jnp.float32
    for bimg in range(BIMG):
        xm = x_ref[bimg]

        acc1 = jnp.dot(xm, w1_ref[...], preferred_element_type=f32)
        o1 = jnp.maximum(acc1 + b1_ref[...], 0.0).astype(jnp.bfloat16)

        for ph in range(4):
            a, b = ph // 2, ph % 2
            src = ((a + 1) % 2) * 2 + ((b + 1) % 2)
            o1p = jax.lax.dot_general(pp_ref[ph],
                                      o1[src * NPIX:(src + 1) * NPIX],
                                      ta, preferred_element_type=f32)
            o1_scr[bimg, pl.ds(ph * PHROWS, PHROWS), :] = (
                o1p.astype(jnp.bfloat16))

        cols = []
        for kh in range(3):
            for kw in range(3):
                ph = (kh % 2) * 2 + (kw % 2)
                base = ph * PHROWS + (kh // 2) * HPP + (kw // 2)
                cols.append(o1_scr[bimg, pl.ds(base, MROWS), :])
        patches = jnp.concatenate(cols, axis=1)
        acc2 = jnp.dot(patches, w2_ref[...], preferred_element_type=f32)
        o2 = jnp.maximum(acc2 + b2_ref[...], 0.0).astype(jnp.bfloat16)
        o2c = jax.lax.dot_general(
            sc_ref[...], o2, ta,
            preferred_element_type=f32).astype(jnp.bfloat16)

        xs = xm[0:NPIX, :]
        acc3 = jnp.dot(o2c, w3_ref[...], preferred_element_type=f32)
        accs = jnp.dot(xs, ws_ref[...], preferred_element_type=f32)
        o3 = jnp.maximum(acc3 + accs + bsum_ref[...], 0.0)

        o_ref[bimg] = jnp.transpose(o3).astype(jnp.bfloat16)


def kernel(x, w1_p, b1_p, w2_p, b2_p, w3_p, b3_p, ws_p, bs_p):
    n = x.shape[0]
    ppmat, sc = _consts()
    xr = x.astype(jnp.bfloat16).reshape(n, CIN, HO, 2, WO, 2)
    xr = jnp.transpose(xr, (0, 3, 5, 2, 4, 1))
    xrows = xr.reshape(n, HW, CIN)

    w1 = w1_p.astype(jnp.bfloat16)
    w2 = w2_p.astype(jnp.bfloat16)
    w3 = w3_p.astype(jnp.bfloat16)
    ws = ws_p.astype(jnp.bfloat16)
    b1 = b1_p.reshape(1, PL)
    b2 = b2_p.reshape(1, PL)
    bsum = (b3_p + bs_p).reshape(1, COUT)

    flops = 2 * n * (HW * CIN * PL + 4 * NPIX * PHROWS * PL
                     + MROWS * 9 * PL * PL + MROWS * MOUT * PL
                     + MOUT * PL * COUT + MOUT * CIN * COUT)
    bytes_accessed = (n * HW * CIN * 2 + n * COUT * MOUT * 4
                      + (w1.size + w2.size + w3.size + ws.size) * 2)
    const = lambda i: (0, 0)
    out = pl.pallas_call(
        _body,
        grid=(n // BIMG,),
        in_specs=[
            pl.BlockSpec((BIMG, HW, CIN), lambda i: (i, 0, 0)),
            pl.BlockSpec((CIN, PL), const),
            pl.BlockSpec((1, PL), const),
            pl.BlockSpec((9 * PL, PL), const),
            pl.BlockSpec((1, PL), const),
            pl.BlockSpec((PL, COUT), const),
            pl.BlockSpec((CIN, COUT), const),
            pl.BlockSpec((1, COUT), const),
            pl.BlockSpec((4, NPIX, PHROWS), lambda i: (0, 0, 0)),
            pl.BlockSpec((MROWS, MOUT), const),
        ],
        out_specs=pl.BlockSpec((BIMG, COUT, MOUT), lambda i: (i, 0, 0)),
        out_shape=jax.ShapeDtypeStruct((n, COUT, MOUT), jnp.bfloat16),
        scratch_shapes=[pltpu.VMEM((BIMG, NROWS, PL), jnp.bfloat16)],
        compiler_params=pltpu.CompilerParams(
            dimension_semantics=("parallel",),
            vmem_limit_bytes=64 * 1024 * 1024),
        cost_estimate=pl.CostEstimate(flops=flops, transcendentals=0,
                                      bytes_accessed=bytes_accessed),
    )(xrows, w1, b1, w2, b2, w3, ws, bsum, ppmat, sc)
    return out.astype(jnp.float32).reshape(n, COUT, HO, WO)


# speedup vs baseline: 1.2428x; 1.0024x over previous
import functools

import jax
import jax.numpy as jnp
import numpy as np
from jax.experimental import pallas as pl
from jax.experimental.pallas import tpu as pltpu

CIN = 256
PL = 128
COUT = 512
H = W = 28
HW = H * W
HO = WO = 14
NPIX = HO * WO
HPP = 15
PHROWS = 16 * HPP
NROWS = 4 * PHROWS
MROWS = HPP * HPP
MOUT = HO * WO
BIMG = 8


def _consts():
    pp = np.zeros((4, NPIX, PHROWS), np.float32)
    for ph in range(4):
        a, b = ph // 2, ph % 2
        for i in range(HPP):
            for j in range(HPP):
                isrc, jsrc = i - (1 - a), j - (1 - b)
                if 0 <= isrc < HO and 0 <= jsrc < WO:
                    pp[ph, isrc * WO + jsrc, HPP * i + j] = 1.0
    sc = np.zeros((MROWS, MOUT), np.float32)
    for i in range(HO):
        for j in range(WO):
            sc[HPP * i + j, i * WO + j] = 1.0
    return jnp.asarray(pp, jnp.bfloat16), jnp.asarray(sc, jnp.bfloat16)


def _body(x_ref, w1_ref, b1_ref, w2_ref, b2_ref, w3_ref, ws_ref, bsum_ref,
          pp_ref, sc_ref, o_ref, o1_scr):
    ta = (((0,), (0,)), ((), ()))
    f32 = jnp.float32
    for bimg in range(BIMG):
        xm = x_ref[bimg]

        acc1 = jnp.dot(xm, w1_ref[...], preferred_element_type=f32)
        o1 = jnp.maximum(acc1 + b1_ref[...], 0.0).astype(jnp.bfloat16)

        for ph in range(4):
            a, b = ph // 2, ph % 2
            src = ((a + 1) % 2) * 2 + ((b + 1) % 2)
            o1p = jax.lax.dot_general(pp_ref[ph],
                                      o1[src * NPIX:(src + 1) * NPIX],
                                      ta, preferred_element_type=f32)
            o1_scr[bimg, pl.ds(ph * PHROWS, PHROWS), :] = (
                o1p.astype(jnp.bfloat16))

        cols = []
        for kh in range(3):
            for kw in range(3):
                ph = (kh % 2) * 2 + (kw % 2)
                base = ph * PHROWS + (kh // 2) * HPP + (kw // 2)
                cols.append(o1_scr[bimg, pl.ds(base, MROWS), :])
        patches = jnp.concatenate(cols, axis=1)
        acc2 = jnp.dot(patches, w2_ref[...], preferred_element_type=f32)
        o2 = jnp.maximum(acc2 + b2_ref[...], 0.0).astype(jnp.bfloat16)
        o2c = jax.lax.dot_general(
            sc_ref[...], o2, ta,
            preferred_element_type=f32).astype(jnp.bfloat16)

        xs = xm[0:NPIX, :]
        acc3 = jnp.dot(o2c, w3_ref[...], preferred_element_type=f32)
        accs = jnp.dot(xs, ws_ref[...], preferred_element_type=f32)
        o3 = jnp.maximum(acc3 + accs + bsum_ref[...], 0.0)

        o_ref[bimg] = jnp.transpose(o3).astype(jnp.bfloat16)


def kernel(x, w1_p, b1_p, w2_p, b2_p, w3_p, b3_p, ws_p, bs_p):
    n = x.shape[0]
    ppmat, sc = _consts()
    xr = x.astype(jnp.bfloat16).reshape(n, CIN, HO, 2, WO, 2)
    xr = jnp.transpose(xr, (0, 3, 5, 2, 4, 1))
    xrows = xr.reshape(n, HW, CIN)

    w1 = w1_p.astype(jnp.bfloat16)
    w2 = w2_p.astype(jnp.bfloat16)
    w3 = w3_p.astype(jnp.bfloat16)
    ws = ws_p.astype(jnp.bfloat16)
    b1 = b1_p.reshape(1, PL)
    b2 = b2_p.reshape(1, PL)
    bsum = (b3_p + bs_p).reshape(1, COUT)

    flops = 2 * n * (HW * CIN * PL + 4 * NPIX * PHROWS * PL
                     + MROWS * 9 * PL * PL + MROWS * MOUT * PL
                     + MOUT * PL * COUT + MOUT * CIN * COUT)
    bytes_accessed = (n * HW * CIN * 2 + n * COUT * MOUT * 4
                      + (w1.size + w2.size + w3.size + ws.size) * 2)
    const = lambda i: (0, 0)
    out = pl.pallas_call(
        _body,
        grid=(n // BIMG,),
        in_specs=[
            pl.BlockSpec((BIMG, HW, CIN), lambda i: (i, 0, 0)),
            pl.BlockSpec((CIN, PL), const),
            pl.BlockSpec((1, PL), const),
            pl.BlockSpec((9 * PL, PL), const),
            pl.BlockSpec((1, PL), const),
            pl.BlockSpec((PL, COUT), const),
            pl.BlockSpec((CIN, COUT), const),
            pl.BlockSpec((1, COUT), const),
            pl.BlockSpec((4, NPIX, PHROWS), lambda i: (0, 0, 0)),
            pl.BlockSpec((MROWS, MOUT), const),
        ],
        out_specs=pl.BlockSpec((BIMG, COUT, MOUT), lambda i: (i, 0, 0)),
        out_shape=jax.ShapeDtypeStruct((n, COUT, MOUT), jnp.bfloat16),
        scratch_shapes=[pltpu.VMEM((BIMG, NROWS, PL), jnp.bfloat16)],
        compiler_params=pltpu.CompilerParams(
            dimension_semantics=("parallel",),
            vmem_limit_bytes=64 * 1024 * 1024),
        cost_estimate=pl.CostEstimate(flops=flops, transcendentals=0,
                                      bytes_accessed=bytes_accessed),
    )(xrows, w1, b1, w2, b2, w3, ws, bsum, ppmat, sc)
    return out.astype(jnp.float32).reshape(n, COUT, HO, WO)
